# baseline (speedup 1.0000x reference)
import jax
import jax.numpy as jnp
from jax import lax
from jax.experimental import pallas as pl
from jax.experimental.pallas import tpu as pltpu
from jax.experimental.pallas import tpu_sc as plsc

_M = 500000
_D = 64
_NSRC = 131072
_NE = _NSRC * _D
_NTILE = 16
_NCHUNK = 20
_RCHUNK = _M // _NCHUNK
_CE = _RCHUNK * _D
_INIT_SLICE = _CE // _NTILE
_PIECE = 4000
_NPIECE = _INIT_SLICE // _PIECE
_SCAN_B = 4096
_EPT = _NE // _NTILE
_NBATCH = _EPT // _SCAN_B
_GRP = 8
_NGRP = _SCAN_B // 16 // _GRP
_REG = 336
_CAP = _GRP * _REG
_FTH = _REG - 16


def _body(inp_hbm, idx_hbm, src_hbm, out_hbm,
          acc, zeros_sp, stage_idx, stage_src, offbuf, valbuf, stage_io,
          sem_in, sem_io):
    cid = lax.axis_index("c")
    sid = lax.axis_index("s")
    iota = lax.iota(jnp.int32, 16)

    def init_const(v, _):
        base = v * 16
        offbuf[pl.ds(base, 16)] = iota + base
        valbuf[pl.ds(base, 16)] = jnp.zeros((16,), jnp.int32)
        return 0

    lax.fori_loop(0, _CAP // 16, init_const, 0, unroll=4)

    @pl.when(sid == 0)
    def _():
        pltpu.sync_copy(valbuf, zeros_sp)

    plsc.subcore_barrier()

    def in_copies(b):
        p = lax.rem(b, 2)
        sbase = sid * _EPT + b * _SCAN_B
        return (
            pltpu.make_async_copy(idx_hbm.at[pl.ds(sbase, _SCAN_B)],
                                  stage_idx.at[pl.ds(p * _SCAN_B, _SCAN_B)],
                                  sem_in.at[p]),
            pltpu.make_async_copy(src_hbm.at[pl.ds(sbase, _SCAN_B)],
                                  stage_src.at[pl.ds(p * _SCAN_B, _SCAN_B)],
                                  sem_in.at[p]),
        )

    def chunk_body(gg, _):
        g = cid * (_NCHUNK // 2) + gg
        ebase = g * _CE
        sbase = sid * _EPT

        tbase = ebase + sid * _INIT_SLICE

        def init_piece(i, _):
            o = i * _PIECE
            pltpu.sync_copy(inp_hbm.at[pl.ds(tbase + o, _PIECE)],
                            stage_io.at[pl.ds(0, _PIECE)])
            pltpu.sync_copy(stage_io.at[pl.ds(0, _PIECE)],
                            acc.at[pl.ds(sid * _INIT_SLICE + o, _PIECE)])
            return 0

        lax.fori_loop(0, _NPIECE, init_piece, 0)
        plsc.subcore_barrier()

        colmb = [iota + (jj * 16 - ebase) for jj in range(4)]

        def flush():
            pltpu.sync_copy(valbuf, acc.at[offbuf], add=True)
            pltpu.sync_copy(zeros_sp, valbuf)

        def batch_body(b, fills):
            for c in in_copies(b):
                c.wait()

            @pl.when(b + 1 < _NBATCH)
            def _():
                for c in in_copies(b + 1):
                    c.start()

            vbase = lax.rem(b, 2) * _SCAN_B

            f15 = jnp.full((16,), 15, jnp.int32)
            neg1 = jnp.full((16,), -1, jnp.int32)

            def group_body(grp, carry):
                fills, pending = carry
                full = pending >= _FTH
                pl.when(full)(flush)
                fills = [jnp.where(full, neg1, f) for f in fills]
                base = vbase + grp * (16 * _GRP)
                rows_l = [stage_idx[pl.ds(base + j * 16, 16)]
                          for j in range(_GRP)]
                vals_l = [stage_src[pl.ds(base + j * 16, 16)]
                          for j in range(_GRP)]
                local_l = [(rows_l[j] << 6) + colmb[j % 4]
                           for j in range(_GRP)]
                mask_l = [local_l[j].astype(jnp.uint32) < jnp.uint32(_CE)
                          for j in range(_GRP)]
                cs_l = [jnp.cumsum(jnp.where(mask_l[j], 1, 0))
                        for j in range(_GRP)]
                pos0_l = [cs_l[j] + fills[j] for j in range(_GRP)]
                new_fills = [pos0_l[j].at[f15].get(mode="promise_in_bounds")
                             for j in range(_GRP)]
                for j in range(_GRP):
                    pos = pos0_l[j] + (j * _REG)
                    plsc.store_scatter(offbuf, [pos], local_l[j],
                                       mask=mask_l[j])
                    plsc.store_scatter(valbuf, [pos], vals_l[j],
                                       mask=mask_l[j])
                m = new_fills[0]
                for k in range(1, _GRP):
                    m = jnp.maximum(m, new_fills[k])
                return new_fills, jnp.max(m)

            return lax.fori_loop(0, _NGRP, group_body, fills)

        for c in in_copies(0):
            c.start()
        lax.fori_loop(0, _NBATCH, batch_body,
                      ([jnp.full((16,), -1, jnp.int32)] * _GRP,
                       jnp.int32(0)))
        flush()
        plsc.subcore_barrier()

        def wb_piece(i, _):
            o = i * _PIECE
            pltpu.sync_copy(acc.at[pl.ds(sid * _INIT_SLICE + o, _PIECE)],
                            stage_io.at[pl.ds(0, _PIECE)])
            pltpu.sync_copy(stage_io.at[pl.ds(0, _PIECE)],
                            out_hbm.at[pl.ds(tbase + o, _PIECE)])
            return 0

        lax.fori_loop(0, _NPIECE, wb_piece, 0)
        plsc.subcore_barrier()
        return 0

    lax.fori_loop(0, _NCHUNK // 2, chunk_body, 0)


def kernel(input, index, src):
    mesh = plsc.VectorSubcoreMesh(core_axis_name="c", subcore_axis_name="s")
    k = pl.kernel(
        _body,
        out_type=jax.ShapeDtypeStruct((_M * _D,), jnp.int32),
        mesh=mesh,
        compiler_params=pltpu.CompilerParams(needs_layout_passes=False),
        scratch_types=[
            pltpu.VMEM_SHARED((_CE,), jnp.int32),
            pltpu.VMEM_SHARED((_CAP,), jnp.int32),
            pltpu.VMEM((2 * _SCAN_B,), jnp.int32),
            pltpu.VMEM((2 * _SCAN_B,), jnp.int32),
            pltpu.VMEM((_CAP,), jnp.int32),
            pltpu.VMEM((_CAP,), jnp.int32),
            pltpu.VMEM((_PIECE,), jnp.int32),
            pltpu.SemaphoreType.DMA((2,)),
            pltpu.SemaphoreType.DMA((2,)),
        ],
    )
    out = k(input.reshape(-1), index.reshape(-1), src.reshape(-1))
    return out.reshape(_M, _D)


# speedup vs baseline: 1.1916x; 1.1916x over previous
import jax
import jax.numpy as jnp
from jax import lax
from jax.experimental import pallas as pl
from jax.experimental.pallas import tpu as pltpu
from jax.experimental.pallas import tpu_sc as plsc

_M = 500000
_D = 64
_NSRC = 131072
_NE = _NSRC * _D
_NTILE = 16
_NCHUNK = 20
_RCHUNK = _M // _NCHUNK
_CE = _RCHUNK * _D
_INIT_SLICE = _CE // _NTILE
_PIECE = 4000
_NPIECE = _INIT_SLICE // _PIECE
_SCAN_B = 4096
_EPT = _NE // _NTILE
_NBATCH = _EPT // _SCAN_B
_GRP = 8
_NGRP = _SCAN_B // 16 // _GRP
_REG = 336
_CAP = _GRP * _REG
_FTH = _REG - 32


def _body(inp_hbm, idx_hbm, src_hbm, out_hbm,
          acc, zeros_sp, stage_idx, stage_src, offbuf, valbuf, stage_io,
          sem_in, sem_io):
    cid = lax.axis_index("c")
    sid = lax.axis_index("s")
    iota = lax.iota(jnp.int32, 16)

    def init_const(v, _):
        base = v * 16
        offbuf[pl.ds(base, 16)] = iota + base
        valbuf[pl.ds(base, 16)] = jnp.zeros((16,), jnp.int32)
        return 0

    lax.fori_loop(0, _CAP // 16, init_const, 0, unroll=4)

    @pl.when(sid == 0)
    def _():
        pltpu.sync_copy(valbuf, zeros_sp)

    plsc.subcore_barrier()

    def in_copies(b):
        p = lax.rem(b, 2)
        sbase = sid * _EPT + b * _SCAN_B
        return (
            pltpu.make_async_copy(idx_hbm.at[pl.ds(sbase, _SCAN_B)],
                                  stage_idx.at[pl.ds(p * _SCAN_B, _SCAN_B)],
                                  sem_in.at[p]),
            pltpu.make_async_copy(src_hbm.at[pl.ds(sbase, _SCAN_B)],
                                  stage_src.at[pl.ds(p * _SCAN_B, _SCAN_B)],
                                  sem_in.at[p]),
        )

    def chunk_body(gg, _):
        g = cid * (_NCHUNK // 2) + gg
        ebase = g * _CE
        sbase = sid * _EPT

        tbase = ebase + sid * _INIT_SLICE

        def init_piece(i, _):
            o = i * _PIECE
            pltpu.sync_copy(inp_hbm.at[pl.ds(tbase + o, _PIECE)],
                            stage_io.at[pl.ds(0, _PIECE)])
            pltpu.sync_copy(stage_io.at[pl.ds(0, _PIECE)],
                            acc.at[pl.ds(sid * _INIT_SLICE + o, _PIECE)])
            return 0

        lax.fori_loop(0, _NPIECE, init_piece, 0)
        plsc.subcore_barrier()

        colmb = [iota + (jj * 16 - ebase) for jj in range(4)]

        def flush():
            pltpu.sync_copy(valbuf, acc.at[offbuf], add=True)
            pltpu.sync_copy(zeros_sp, valbuf)

        def batch_body(b, fills):
            for c in in_copies(b):
                c.wait()

            @pl.when(b + 1 < _NBATCH)
            def _():
                for c in in_copies(b + 1):
                    c.start()

            vbase = lax.rem(b, 2) * _SCAN_B

            f15 = jnp.full((16,), 15, jnp.int32)
            neg1 = jnp.full((16,), -1, jnp.int32)

            def group_body(grp, carry):
                fills, pending = carry
                full = pending >= _FTH
                pl.when(full)(flush)
                fills = [jnp.where(full, neg1, f) for f in fills]
                m = fills[0]
                for k in range(1, _GRP):
                    m = jnp.maximum(m, fills[k])
                pending_next = jnp.max(m)
                base = vbase + grp * (16 * _GRP)
                rows_l = [stage_idx[pl.ds(base + j * 16, 16)]
                          for j in range(_GRP)]
                vals_l = [stage_src[pl.ds(base + j * 16, 16)]
                          for j in range(_GRP)]
                local_l = [(rows_l[j] << 6) + colmb[j % 4]
                           for j in range(_GRP)]
                mask_l = [local_l[j].astype(jnp.uint32) < jnp.uint32(_CE)
                          for j in range(_GRP)]
                cs_l = [jnp.cumsum(jnp.where(mask_l[j], 1, 0))
                        for j in range(_GRP)]
                pos0_l = [cs_l[j] + fills[j] for j in range(_GRP)]
                new_fills = [pos0_l[j].at[f15].get(mode="promise_in_bounds")
                             for j in range(_GRP)]
                for j in range(_GRP):
                    pos = pos0_l[j] + (j * _REG)
                    plsc.store_scatter(offbuf, [pos], local_l[j],
                                       mask=mask_l[j])
                    plsc.store_scatter(valbuf, [pos], vals_l[j],
                                       mask=mask_l[j])
                return new_fills, pending_next

            return lax.fori_loop(0, _NGRP, group_body, fills)

        for c in in_copies(0):
            c.start()
        lax.fori_loop(0, _NBATCH, batch_body,
                      ([jnp.full((16,), -1, jnp.int32)] * _GRP,
                       jnp.int32(0)))
        flush()
        plsc.subcore_barrier()

        def wb_piece(i, _):
            o = i * _PIECE
            pltpu.sync_copy(acc.at[pl.ds(sid * _INIT_SLICE + o, _PIECE)],
                            stage_io.at[pl.ds(0, _PIECE)])
            pltpu.sync_copy(stage_io.at[pl.ds(0, _PIECE)],
                            out_hbm.at[pl.ds(tbase + o, _PIECE)])
            return 0

        lax.fori_loop(0, _NPIECE, wb_piece, 0)
        plsc.subcore_barrier()
        return 0

    lax.fori_loop(0, _NCHUNK // 2, chunk_body, 0)


def kernel(input, index, src):
    mesh = plsc.VectorSubcoreMesh(core_axis_name="c", subcore_axis_name="s")
    k = pl.kernel(
        _body,
        out_type=jax.ShapeDtypeStruct((_M * _D,), jnp.int32),
        mesh=mesh,
        compiler_params=pltpu.CompilerParams(needs_layout_passes=False),
        scratch_types=[
            pltpu.VMEM_SHARED((_CE,), jnp.int32),
            pltpu.VMEM_SHARED((_CAP,), jnp.int32),
            pltpu.VMEM((2 * _SCAN_B,), jnp.int32),
            pltpu.VMEM((2 * _SCAN_B,), jnp.int32),
            pltpu.VMEM((_CAP,), jnp.int32),
            pltpu.VMEM((_CAP,), jnp.int32),
            pltpu.VMEM((_PIECE,), jnp.int32),
            pltpu.SemaphoreType.DMA((2,)),
            pltpu.SemaphoreType.DMA((2,)),
        ],
    )
    out = k(input.reshape(-1), index.reshape(-1), src.reshape(-1))
    return out.reshape(_M, _D)


# speedup vs baseline: 1.2389x; 1.0397x over previous
import jax
import jax.numpy as jnp
from jax import lax
from jax.experimental import pallas as pl
from jax.experimental.pallas import tpu as pltpu
from jax.experimental.pallas import tpu_sc as plsc

_M = 500000
_D = 64
_NSRC = 131072
_NE = _NSRC * _D
_NTILE = 16
_NCHUNK = 20
_RCHUNK = _M // _NCHUNK
_CE = _RCHUNK * _D
_INIT_SLICE = _CE // _NTILE
_PIECE = 4000
_NPIECE = _INIT_SLICE // _PIECE
_SCAN_B = 4096
_EPT = _NE // _NTILE
_NBATCH = _EPT // _SCAN_B
_GRP = 8
_NGRP = _SCAN_B // 16 // _GRP
_REG = 336
_CAP = _GRP * _REG
_FTH = _REG - 32


def _body(inp_hbm, idx_hbm, src_hbm, out_hbm,
          acc, zeros_sp, stage_idx, stage_src, offbuf, valbuf, stage_io,
          sem_in, sem_io):
    cid = lax.axis_index("c")
    sid = lax.axis_index("s")
    iota = lax.iota(jnp.int32, 16)

    def init_const(v, _):
        base = v * 16
        offbuf[pl.ds(base, 16)] = iota + base
        valbuf[pl.ds(base, 16)] = jnp.zeros((16,), jnp.int32)
        return 0

    lax.fori_loop(0, _CAP // 16, init_const, 0, unroll=4)

    @pl.when(sid == 0)
    def _():
        pltpu.sync_copy(valbuf, zeros_sp)

    plsc.subcore_barrier()

    def in_copies(b):
        p = lax.rem(b, 2)
        sbase = sid * _EPT + b * _SCAN_B
        return (
            pltpu.make_async_copy(idx_hbm.at[pl.ds(sbase, _SCAN_B)],
                                  stage_idx.at[pl.ds(p * _SCAN_B, _SCAN_B)],
                                  sem_in.at[p]),
            pltpu.make_async_copy(src_hbm.at[pl.ds(sbase, _SCAN_B)],
                                  stage_src.at[pl.ds(p * _SCAN_B, _SCAN_B)],
                                  sem_in.at[p]),
        )

    def chunk_body(gg, _):
        g = cid * (_NCHUNK // 2) + gg
        ebase = g * _CE
        sbase = sid * _EPT

        tbase = ebase + sid * _INIT_SLICE

        def rd_copy(i):
            p = lax.rem(i, 2)
            return pltpu.make_async_copy(
                inp_hbm.at[pl.ds(tbase + i * _PIECE, _PIECE)],
                stage_io.at[pl.ds(p * _PIECE, _PIECE)], sem_io.at[p])

        rd_copy(0).start()

        def init_piece(i, _):
            rd_copy(i).wait()

            @pl.when(i + 1 < _NPIECE)
            def _():
                rd_copy(i + 1).start()

            p = lax.rem(i, 2)
            pltpu.sync_copy(
                stage_io.at[pl.ds(p * _PIECE, _PIECE)],
                acc.at[pl.ds(sid * _INIT_SLICE + i * _PIECE, _PIECE)])
            return 0

        lax.fori_loop(0, _NPIECE, init_piece, 0)
        plsc.subcore_barrier()

        colmb = [iota + (jj * 16 - ebase) for jj in range(4)]

        def flush():
            pltpu.sync_copy(valbuf, acc.at[offbuf], add=True)
            pltpu.sync_copy(zeros_sp, valbuf)

        def batch_body(b, fills):
            for c in in_copies(b):
                c.wait()

            @pl.when(b + 1 < _NBATCH)
            def _():
                for c in in_copies(b + 1):
                    c.start()

            vbase = lax.rem(b, 2) * _SCAN_B

            f15 = jnp.full((16,), 15, jnp.int32)
            neg1 = jnp.full((16,), -1, jnp.int32)

            def group_body(grp, carry):
                fills, pending = carry
                full = pending >= _FTH
                pl.when(full)(flush)
                fills = [jnp.where(full, neg1, f) for f in fills]
                m = fills[0]
                for k in range(1, _GRP):
                    m = jnp.maximum(m, fills[k])
                pending_next = jnp.max(m)
                base = vbase + grp * (16 * _GRP)
                rows_l = [stage_idx[pl.ds(base + j * 16, 16)]
                          for j in range(_GRP)]
                vals_l = [stage_src[pl.ds(base + j * 16, 16)]
                          for j in range(_GRP)]
                local_l = [(rows_l[j] << 6) + colmb[j % 4]
                           for j in range(_GRP)]
                mask_l = [local_l[j].astype(jnp.uint32) < jnp.uint32(_CE)
                          for j in range(_GRP)]
                cs_l = [jnp.cumsum(jnp.where(mask_l[j], 1, 0))
                        for j in range(_GRP)]
                pos0_l = [cs_l[j] + fills[j] for j in range(_GRP)]
                new_fills = [pos0_l[j].at[f15].get(mode="promise_in_bounds")
                             for j in range(_GRP)]
                for j in range(_GRP):
                    pos = pos0_l[j] + (j * _REG)
                    plsc.store_scatter(offbuf, [pos], local_l[j],
                                       mask=mask_l[j])
                    plsc.store_scatter(valbuf, [pos], vals_l[j],
                                       mask=mask_l[j])
                return new_fills, pending_next

            return lax.fori_loop(0, _NGRP, group_body, fills)

        for c in in_copies(0):
            c.start()
        lax.fori_loop(0, _NBATCH, batch_body,
                      ([jnp.full((16,), -1, jnp.int32)] * _GRP,
                       jnp.int32(0)))
        flush()
        plsc.subcore_barrier()

        def wr_copy(i):
            p = lax.rem(i, 2)
            return pltpu.make_async_copy(
                stage_io.at[pl.ds(p * _PIECE, _PIECE)],
                out_hbm.at[pl.ds(tbase + i * _PIECE, _PIECE)], sem_io.at[p])

        def wb_piece(i, _):
            pl.when(i >= 2)(lambda: wr_copy(i - 2).wait())
            p = lax.rem(i, 2)
            pltpu.sync_copy(
                acc.at[pl.ds(sid * _INIT_SLICE + i * _PIECE, _PIECE)],
                stage_io.at[pl.ds(p * _PIECE, _PIECE)])
            wr_copy(i).start()
            return 0

        lax.fori_loop(0, _NPIECE, wb_piece, 0)
        wr_copy(_NPIECE - 2).wait()
        wr_copy(_NPIECE - 1).wait()
        plsc.subcore_barrier()
        return 0

    lax.fori_loop(0, _NCHUNK // 2, chunk_body, 0)


def kernel(input, index, src):
    mesh = plsc.VectorSubcoreMesh(core_axis_name="c", subcore_axis_name="s")
    k = pl.kernel(
        _body,
        out_type=jax.ShapeDtypeStruct((_M * _D,), jnp.int32),
        mesh=mesh,
        compiler_params=pltpu.CompilerParams(needs_layout_passes=False),
        scratch_types=[
            pltpu.VMEM_SHARED((_CE,), jnp.int32),
            pltpu.VMEM_SHARED((_CAP,), jnp.int32),
            pltpu.VMEM((2 * _SCAN_B,), jnp.int32),
            pltpu.VMEM((2 * _SCAN_B,), jnp.int32),
            pltpu.VMEM((_CAP,), jnp.int32),
            pltpu.VMEM((_CAP,), jnp.int32),
            pltpu.VMEM((2 * _PIECE,), jnp.int32),
            pltpu.SemaphoreType.DMA((2,)),
            pltpu.SemaphoreType.DMA((2,)),
        ],
    )
    out = k(input.reshape(-1), index.reshape(-1), src.reshape(-1))
    return out.reshape(_M, _D)


# speedup vs baseline: 1.2413x; 1.0020x over previous
import jax
import jax.numpy as jnp
from jax import lax
from jax.experimental import pallas as pl
from jax.experimental.pallas import tpu as pltpu
from jax.experimental.pallas import tpu_sc as plsc

_M = 500000
_D = 64
_NSRC = 131072
_NE = _NSRC * _D
_NTILE = 16
_NCHUNK = 20
_RCHUNK = _M // _NCHUNK
_CE = _RCHUNK * _D
_INIT_SLICE = _CE // _NTILE
_PIECE = 4000
_NPIECE = _INIT_SLICE // _PIECE
_SCAN_B = 4096
_EPT = _NE // _NTILE
_NBATCH = _EPT // _SCAN_B
_GRP = 16
_NGRP = _SCAN_B // 16 // _GRP
_REG = 168
_CAP = _GRP * _REG
_FTH = _REG - 32


def _body(inp_hbm, idx_hbm, src_hbm, out_hbm,
          acc, zeros_sp, stage_idx, stage_src, offbuf, valbuf, stage_io,
          sem_in, sem_io):
    cid = lax.axis_index("c")
    sid = lax.axis_index("s")
    iota = lax.iota(jnp.int32, 16)

    def init_const(v, _):
        base = v * 16
        offbuf[pl.ds(base, 16)] = iota + base
        valbuf[pl.ds(base, 16)] = jnp.zeros((16,), jnp.int32)
        return 0

    lax.fori_loop(0, _CAP // 16, init_const, 0, unroll=4)

    @pl.when(sid == 0)
    def _():
        pltpu.sync_copy(valbuf, zeros_sp)

    plsc.subcore_barrier()

    def in_copies(b):
        p = lax.rem(b, 2)
        sbase = sid * _EPT + b * _SCAN_B
        return (
            pltpu.make_async_copy(idx_hbm.at[pl.ds(sbase, _SCAN_B)],
                                  stage_idx.at[pl.ds(p * _SCAN_B, _SCAN_B)],
                                  sem_in.at[p]),
            pltpu.make_async_copy(src_hbm.at[pl.ds(sbase, _SCAN_B)],
                                  stage_src.at[pl.ds(p * _SCAN_B, _SCAN_B)],
                                  sem_in.at[p]),
        )

    def chunk_body(gg, _):
        g = cid * (_NCHUNK // 2) + gg
        ebase = g * _CE
        sbase = sid * _EPT

        tbase = ebase + sid * _INIT_SLICE

        def rd_copy(i):
            p = lax.rem(i, 2)
            return pltpu.make_async_copy(
                inp_hbm.at[pl.ds(tbase + i * _PIECE, _PIECE)],
                stage_io.at[pl.ds(p * _PIECE, _PIECE)], sem_io.at[p])

        rd_copy(0).start()

        def init_piece(i, _):
            rd_copy(i).wait()

            @pl.when(i + 1 < _NPIECE)
            def _():
                rd_copy(i + 1).start()

            p = lax.rem(i, 2)
            pltpu.sync_copy(
                stage_io.at[pl.ds(p * _PIECE, _PIECE)],
                acc.at[pl.ds(sid * _INIT_SLICE + i * _PIECE, _PIECE)])
            return 0

        lax.fori_loop(0, _NPIECE, init_piece, 0)
        plsc.subcore_barrier()

        colmb = [iota + (jj * 16 - ebase) for jj in range(4)]

        def flush():
            pltpu.sync_copy(valbuf, acc.at[offbuf], add=True)
            pltpu.sync_copy(zeros_sp, valbuf)

        def batch_body(b, fills):
            for c in in_copies(b):
                c.wait()

            @pl.when(b + 1 < _NBATCH)
            def _():
                for c in in_copies(b + 1):
                    c.start()

            vbase = lax.rem(b, 2) * _SCAN_B

            f15 = jnp.full((16,), 15, jnp.int32)
            neg1 = jnp.full((16,), -1, jnp.int32)

            def group_body(grp, carry):
                fills, pending = carry
                full = pending >= _FTH
                pl.when(full)(flush)
                fills = [jnp.where(full, neg1, f) for f in fills]
                m = fills[0]
                for k in range(1, _GRP):
                    m = jnp.maximum(m, fills[k])
                pending_next = jnp.max(m)
                base = vbase + grp * (16 * _GRP)
                rows_l = [stage_idx[pl.ds(base + j * 16, 16)]
                          for j in range(_GRP)]
                vals_l = [stage_src[pl.ds(base + j * 16, 16)]
                          for j in range(_GRP)]
                local_l = [(rows_l[j] << 6) + colmb[j % 4]
                           for j in range(_GRP)]
                mask_l = [local_l[j].astype(jnp.uint32) < jnp.uint32(_CE)
                          for j in range(_GRP)]
                cs_l = [jnp.cumsum(jnp.where(mask_l[j], 1, 0))
                        for j in range(_GRP)]
                pos0_l = [cs_l[j] + fills[j] for j in range(_GRP)]
                new_fills = [pos0_l[j].at[f15].get(mode="promise_in_bounds")
                             for j in range(_GRP)]
                for j in range(_GRP):
                    pos = pos0_l[j] + (j * _REG)
                    plsc.store_scatter(offbuf, [pos], local_l[j],
                                       mask=mask_l[j])
                    plsc.store_scatter(valbuf, [pos], vals_l[j],
                                       mask=mask_l[j])
                return new_fills, pending_next

            return lax.fori_loop(0, _NGRP, group_body, fills)

        for c in in_copies(0):
            c.start()
        lax.fori_loop(0, _NBATCH, batch_body,
                      ([jnp.full((16,), -1, jnp.int32)] * _GRP,
                       jnp.int32(0)))
        flush()
        plsc.subcore_barrier()

        def wr_copy(i):
            p = lax.rem(i, 2)
            return pltpu.make_async_copy(
                stage_io.at[pl.ds(p * _PIECE, _PIECE)],
                out_hbm.at[pl.ds(tbase + i * _PIECE, _PIECE)], sem_io.at[p])

        def wb_piece(i, _):
            pl.when(i >= 2)(lambda: wr_copy(i - 2).wait())
            p = lax.rem(i, 2)
            pltpu.sync_copy(
                acc.at[pl.ds(sid * _INIT_SLICE + i * _PIECE, _PIECE)],
                stage_io.at[pl.ds(p * _PIECE, _PIECE)])
            wr_copy(i).start()
            return 0

        lax.fori_loop(0, _NPIECE, wb_piece, 0)
        wr_copy(_NPIECE - 2).wait()
        wr_copy(_NPIECE - 1).wait()
        plsc.subcore_barrier()
        return 0

    lax.fori_loop(0, _NCHUNK // 2, chunk_body, 0)


def kernel(input, index, src):
    mesh = plsc.VectorSubcoreMesh(core_axis_name="c", subcore_axis_name="s")
    k = pl.kernel(
        _body,
        out_type=jax.ShapeDtypeStruct((_M * _D,), jnp.int32),
        mesh=mesh,
        compiler_params=pltpu.CompilerParams(needs_layout_passes=False),
        scratch_types=[
            pltpu.VMEM_SHARED((_CE,), jnp.int32),
            pltpu.VMEM_SHARED((_CAP,), jnp.int32),
            pltpu.VMEM((2 * _SCAN_B,), jnp.int32),
            pltpu.VMEM((2 * _SCAN_B,), jnp.int32),
            pltpu.VMEM((_CAP,), jnp.int32),
            pltpu.VMEM((_CAP,), jnp.int32),
            pltpu.VMEM((2 * _PIECE,), jnp.int32),
            pltpu.SemaphoreType.DMA((2,)),
            pltpu.SemaphoreType.DMA((2,)),
        ],
    )
    out = k(input.reshape(-1), index.reshape(-1), src.reshape(-1))
    return out.reshape(_M, _D)


# speedup vs baseline: 1.2970x; 1.0449x over previous
import jax
import jax.numpy as jnp
from jax import lax
from jax.experimental import pallas as pl
from jax.experimental.pallas import tpu as pltpu
from jax.experimental.pallas import tpu_sc as plsc

_M = 500000
_D = 64
_NSRC = 131072
_NE = _NSRC * _D
_NTILE = 16
_NCHUNK = 20
_RCHUNK = _M // _NCHUNK
_CE = _RCHUNK * _D
_INIT_SLICE = _CE // _NTILE
_PIECE = 4000
_NPIECE = _INIT_SLICE // _PIECE
_SCAN_B = 4096
_EPT = _NE // _NTILE
_NBATCH = _EPT // _SCAN_B
_GRP = 16
_NGRP = _SCAN_B // 16 // _GRP
_REG = 96
_CAP = _GRP * _REG
_CAPT = 2 * _CAP
_FTH = _REG - 32


def _body(inp_hbm, idx_hbm, src_hbm, out_hbm,
          acc, zeros_sp, stage_idx, stage_src, offbuf, valbuf, stage_io,
          sem_in, sem_io, sem_fl):
    cid = lax.axis_index("c")
    sid = lax.axis_index("s")
    iota = lax.iota(jnp.int32, 16)

    def init_const(v, _):
        base = v * 16
        offbuf[pl.ds(base, 16)] = iota + base
        valbuf[pl.ds(base, 16)] = jnp.zeros((16,), jnp.int32)
        return 0

    lax.fori_loop(0, _CAPT // 16, init_const, 0, unroll=4)

    @pl.when(sid == 0)
    def _():
        pltpu.sync_copy(valbuf.at[pl.ds(0, _CAP)], zeros_sp)

    plsc.subcore_barrier()

    def in_copies(b):
        p = lax.rem(b, 2)
        sbase = sid * _EPT + b * _SCAN_B
        return (
            pltpu.make_async_copy(idx_hbm.at[pl.ds(sbase, _SCAN_B)],
                                  stage_idx.at[pl.ds(p * _SCAN_B, _SCAN_B)],
                                  sem_in.at[p]),
            pltpu.make_async_copy(src_hbm.at[pl.ds(sbase, _SCAN_B)],
                                  stage_src.at[pl.ds(p * _SCAN_B, _SCAN_B)],
                                  sem_in.at[p]),
        )

    def chunk_body(gg, _):
        g = cid * (_NCHUNK // 2) + gg
        ebase = g * _CE
        sbase = sid * _EPT

        tbase = ebase + sid * _INIT_SLICE

        def rd_copy(i):
            p = lax.rem(i, 2)
            return pltpu.make_async_copy(
                inp_hbm.at[pl.ds(tbase + i * _PIECE, _PIECE)],
                stage_io.at[pl.ds(p * _PIECE, _PIECE)], sem_io.at[p])

        rd_copy(0).start()

        def init_piece(i, _):
            rd_copy(i).wait()

            @pl.when(i + 1 < _NPIECE)
            def _():
                rd_copy(i + 1).start()

            p = lax.rem(i, 2)
            pltpu.sync_copy(
                stage_io.at[pl.ds(p * _PIECE, _PIECE)],
                acc.at[pl.ds(sid * _INIT_SLICE + i * _PIECE, _PIECE)])
            return 0

        lax.fori_loop(0, _NPIECE, init_piece, 0)
        plsc.subcore_barrier()

        colmb = [iota + (jj * 16 - ebase) for jj in range(4)]

        def fl_copy(h):
            o = h * _CAP
            return pltpu.make_async_copy(
                valbuf.at[pl.ds(o, _CAP)],
                acc.at[offbuf.at[pl.ds(o, _CAP)]], sem_fl)

        def restore(h):
            pltpu.sync_copy(zeros_sp, valbuf.at[pl.ds(h * _CAP, _CAP)])

        def batch_body(b, fills):
            for c in in_copies(b):
                c.wait()

            @pl.when(b + 1 < _NBATCH)
            def _():
                for c in in_copies(b + 1):
                    c.start()

            vbase = lax.rem(b, 2) * _SCAN_B

            f15 = jnp.full((16,), 15, jnp.int32)
            neg1 = jnp.full((16,), -1, jnp.int32)

            def group_body(grp, carry):
                fills, pending, nf = carry
                hc = lax.rem(nf, 2)
                full = pending >= hc * _CAP + _FTH

                @pl.when(full)
                def _():
                    pl.when(nf >= 1)(lambda: fl_copy(1 - hc).wait())
                    restore(1 - hc)
                    fl_copy(hc).start()

                resetv = neg1 + (1 - hc) * _CAP
                nf = jnp.where(full, nf + 1, nf)
                fills = [jnp.where(full, resetv, f) for f in fills]
                m = fills[0]
                for k in range(1, _GRP):
                    m = jnp.maximum(m, fills[k])
                pending_next = jnp.max(m)
                base = vbase + grp * (16 * _GRP)
                rows_l = [stage_idx[pl.ds(base + j * 16, 16)]
                          for j in range(_GRP)]
                vals_l = [stage_src[pl.ds(base + j * 16, 16)]
                          for j in range(_GRP)]
                local_l = [(rows_l[j] << 6) + colmb[j % 4]
                           for j in range(_GRP)]
                mask_l = [local_l[j].astype(jnp.uint32) < jnp.uint32(_CE)
                          for j in range(_GRP)]
                cs_l = [jnp.cumsum(jnp.where(mask_l[j], 1, 0))
                        for j in range(_GRP)]
                pos0_l = [cs_l[j] + fills[j] for j in range(_GRP)]
                new_fills = [pos0_l[j].at[f15].get(mode="promise_in_bounds")
                             for j in range(_GRP)]
                for j in range(_GRP):
                    pos = pos0_l[j] + (j * _REG)
                    plsc.store_scatter(offbuf, [pos], local_l[j],
                                       mask=mask_l[j])
                    plsc.store_scatter(valbuf, [pos], vals_l[j],
                                       mask=mask_l[j])
                return new_fills, pending_next, nf

            return lax.fori_loop(0, _NGRP, group_body, fills)

        for c in in_copies(0):
            c.start()
        _, _, nf = lax.fori_loop(
            0, _NBATCH, batch_body,
            ([jnp.full((16,), -1, jnp.int32)] * _GRP,
             jnp.int32(0), jnp.int32(0)))
        hl = 1 - lax.rem(nf, 2)

        @pl.when(nf >= 1)
        def _():
            fl_copy(hl).wait()
            restore(hl)

        ha = lax.rem(nf, 2)
        pltpu.sync_copy(valbuf.at[pl.ds(ha * _CAP, _CAP)],
                        acc.at[offbuf.at[pl.ds(ha * _CAP, _CAP)]], add=True)
        restore(ha)
        plsc.subcore_barrier()

        def wr_copy(i):
            p = lax.rem(i, 2)
            return pltpu.make_async_copy(
                stage_io.at[pl.ds(p * _PIECE, _PIECE)],
                out_hbm.at[pl.ds(tbase + i * _PIECE, _PIECE)], sem_io.at[p])

        def wb_piece(i, _):
            pl.when(i >= 2)(lambda: wr_copy(i - 2).wait())
            p = lax.rem(i, 2)
            pltpu.sync_copy(
                acc.at[pl.ds(sid * _INIT_SLICE + i * _PIECE, _PIECE)],
                stage_io.at[pl.ds(p * _PIECE, _PIECE)])
            wr_copy(i).start()
            return 0

        lax.fori_loop(0, _NPIECE, wb_piece, 0)
        wr_copy(_NPIECE - 2).wait()
        wr_copy(_NPIECE - 1).wait()
        plsc.subcore_barrier()
        return 0

    lax.fori_loop(0, _NCHUNK // 2, chunk_body, 0)


def kernel(input, index, src):
    mesh = plsc.VectorSubcoreMesh(core_axis_name="c", subcore_axis_name="s")
    k = pl.kernel(
        _body,
        out_type=jax.ShapeDtypeStruct((_M * _D,), jnp.int32),
        mesh=mesh,
        compiler_params=pltpu.CompilerParams(needs_layout_passes=False),
        scratch_types=[
            pltpu.VMEM_SHARED((_CE,), jnp.int32),
            pltpu.VMEM_SHARED((_CAP,), jnp.int32),
            pltpu.VMEM((2 * _SCAN_B,), jnp.int32),
            pltpu.VMEM((2 * _SCAN_B,), jnp.int32),
            pltpu.VMEM((_CAPT,), jnp.int32),
            pltpu.VMEM((_CAPT,), jnp.int32),
            pltpu.VMEM((2 * _PIECE,), jnp.int32),
            pltpu.SemaphoreType.DMA((2,)),
            pltpu.SemaphoreType.DMA((2,)),
            pltpu.SemaphoreType.DMA,
        ],
    )
    out = k(input.reshape(-1), index.reshape(-1), src.reshape(-1))
    return out.reshape(_M, _D)
